# baseline (speedup 1.0000x reference)
import functools

import jax
import jax.numpy as jnp
from jax import lax
from jax.experimental import pallas as pl
from jax.experimental.pallas import tpu as pltpu
from jax.experimental.pallas import tpu_sc as plsc

AXES_LIMIT = 40.0
RESOLUTION = 0.25
WL = int(AXES_LIMIT * 2 / RESOLUTION)
HW = WL * WL

B = 2
C = 128
N = 20000
NPTS = B * N

NTILES = 32
PLANES_PER_TILE = (B * C) // NTILES
CHUNK = 2000
NCHUNK = N // CHUNK
GI = CHUNK // 16
P1SPAN = 2560


def _sc_body(fmap_hbm, loc_hbm, empty_hbm, out_hbm,
             planebuf, idxchunk, outchunk, locbuf, idxmbuf, emptybuf,
             idxm_sh):
    cid = lax.axis_index("c")
    sid = lax.axis_index("s")
    lanes = lax.iota(jnp.int32, 16)

    start = jnp.minimum(sid * P1SPAN, NPTS - P1SPAN)
    pltpu.sync_copy(loc_hbm.at[pl.ds(start * 2, P1SPAN * 2)], locbuf)

    def p1_body(i, _):
        base = i * 32
        xv = plsc.load_gather(locbuf, [lanes * 2 + base])
        yv = plsc.load_gather(locbuf, [lanes * 2 + base + 1])
        m = (xv > -1.0) & (xv < 1.0) & (yv > -1.0) & (yv < 1.0)
        x = jnp.clip(xv, -0.999, 0.999) * AXES_LIMIT
        y = jnp.clip(yv, -0.999, 0.999) * AXES_LIMIT
        row = ((AXES_LIMIT - y) / RESOLUTION).astype(jnp.int32)
        col = ((AXES_LIMIT + x) / RESOLUTION).astype(jnp.int32)
        idx = jnp.where(m, row * WL + col, HW)
        idxmbuf[pl.ds(i * 16, 16)] = idx
        return 0

    lax.fori_loop(0, P1SPAN // 16, p1_body, 0)
    pltpu.sync_copy(idxmbuf, idxm_sh.at[pl.ds(start, P1SPAN)])
    plsc.subcore_barrier()

    pltpu.sync_copy(empty_hbm, emptybuf)
    w = sid * 2 + cid

    def plane_body(j, _):
        p = w * PLANES_PER_TILE + j
        b = p // C
        ch = p % C
        pltpu.sync_copy(fmap_hbm.at[b, ch], planebuf.at[pl.ds(0, HW)])
        evv = plsc.load_gather(emptybuf, [jnp.full((16,), ch, jnp.int32)])
        planebuf[pl.ds(HW, 16)] = evv

        def chunk_body(k, _):
            pltpu.sync_copy(idxm_sh.at[pl.ds(b * N + k * CHUNK, CHUNK)],
                            idxchunk)

            def g_body(i, _):
                iv = idxchunk[pl.ds(i * 16, 16)]
                outchunk[pl.ds(i * 16, 16)] = plsc.load_gather(planebuf, [iv])
                return 0

            lax.fori_loop(0, GI, g_body, 0)
            pltpu.sync_copy(outchunk,
                            out_hbm.at[b, ch, pl.ds(k * CHUNK, CHUNK)])
            return 0

        lax.fori_loop(0, NCHUNK, chunk_body, 0)
        return 0

    lax.fori_loop(0, PLANES_PER_TILE, plane_body, 0)


@functools.partial(
    pl.kernel,
    out_type=jax.ShapeDtypeStruct((B, C, N), jnp.float32),
    mesh=plsc.VectorSubcoreMesh(core_axis_name="c", subcore_axis_name="s"),
    compiler_params=pltpu.CompilerParams(use_tc_tiling_on_sc=False,
                                         needs_layout_passes=False),
    scratch_types=[
        pltpu.VMEM((HW + 16,), jnp.float32),
        pltpu.VMEM((CHUNK,), jnp.int32),
        pltpu.VMEM((CHUNK,), jnp.float32),
        pltpu.VMEM((P1SPAN * 2,), jnp.float32),
        pltpu.VMEM((P1SPAN,), jnp.int32),
        pltpu.VMEM((C,), jnp.float32),
        pltpu.VMEM_SHARED((NPTS,), jnp.int32),
    ],
)
def _sc_gather(fmap_hbm, loc_hbm, empty_hbm, out_hbm, *scratch):
    _sc_body(fmap_hbm, loc_hbm, empty_hbm, out_hbm, *scratch)


def _tr_kernel(x_ref, o_ref):
    o_ref[0] = x_ref[0].T


_transpose = pl.pallas_call(
    _tr_kernel,
    out_shape=jax.ShapeDtypeStruct((B, N, C), jnp.float32),
    grid=(B,),
    in_specs=[pl.BlockSpec((1, C, N), lambda b: (b, 0, 0))],
    out_specs=pl.BlockSpec((1, N, C), lambda b: (b, 0, 0)),
)


def kernel(fmap, loc, empty):
    fmap_r = fmap.reshape(B, C, HW)
    loc_flat = loc.reshape(NPTS * 2)
    out_t = _sc_gather(fmap_r, loc_flat, empty)
    return _transpose(out_t)
